# baseline (speedup 1.0000x reference)
import functools

import jax
import jax.numpy as jnp
from jax.experimental import pallas as pl
from jax.experimental.pallas import tpu as pltpu
from jax.experimental.pallas import tpu_sc as plsc

N = 4096
D = 128
HEADS = 4


def _embed_gather(emb, ids):
    V, Dm = emb.shape
    B = ids.shape[0]
    info = plsc.get_sparse_core_info()
    NC, NS = info.num_cores, info.num_subcores
    NW = NC * NS
    bpw = B // NW
    mesh = plsc.VectorSubcoreMesh(core_axis_name="c", subcore_axis_name="s")

    @functools.partial(
        pl.kernel,
        mesh=mesh,
        out_type=jax.ShapeDtypeStruct((B, Dm), jnp.float32),
        scratch_types=[
            pltpu.VMEM((bpw,), jnp.int32),
            pltpu.VMEM((bpw, Dm), jnp.float32),
            pltpu.SemaphoreType.DMA,
        ],
    )
    def gather_kernel(emb_hbm, ids_hbm, out_hbm, idx_v, rows_v, sem):
        wid = jax.lax.axis_index("s") * NC + jax.lax.axis_index("c")
        base = wid * bpw
        pltpu.sync_copy(ids_hbm.at[pl.ds(base, bpw)], idx_v)
        pltpu.async_copy(emb_hbm.at[idx_v], rows_v, sem).wait()
        pltpu.sync_copy(rows_v, out_hbm.at[pl.ds(base, bpw)])

    return gather_kernel(emb, ids)


def _gat_layer(h, adj, W, Asrc, Adst, dh, concat, residual):
    n, din = h.shape
    hd = HEADS * dh
    dout = hd if concat else dh
    BN = 1024 if concat else 512

    def body(h_ref, adj_ref, W_ref, Asrc_ref, Adst_ref, out_ref,
             Wh_ref, Whz_ref, es_ref, edT_ref):
        i = pl.program_id(0)

        @pl.when(i == 0)
        def _():
            Wh = jnp.dot(h_ref[...], W_ref[...],
                         preferred_element_type=jnp.float32)
            Wh_ref[...] = Wh
            for g in range(HEADS):
                Whz_ref[g] = jnp.concatenate(
                    [Wh[:, g * dh:(g + 1) * dh].astype(jnp.bfloat16),
                     jnp.ones((n, 1), jnp.bfloat16)], axis=1)
            src = jnp.dot(Wh, Asrc_ref[...],
                          preferred_element_type=jnp.float32)
            dstT = jnp.dot(Wh, Adst_ref[...],
                           preferred_element_type=jnp.float32).T
            es_ref[...] = jnp.exp2(
                jnp.concatenate([src, 0.2 * src], axis=1))
            edT_ref[...] = jnp.exp2(
                jnp.concatenate([dstT, 0.2 * dstT], axis=0))

        adj = adj_ref[...]
        esB = es_ref[pl.ds(i * BN, BN), :]
        outs = []
        for hh in range(HEADS):
            u1 = (esB[:, hh:hh + 1]
                  * edT_ref[hh:hh + 1, :]).astype(jnp.bfloat16)
            u2 = (esB[:, HEADS + hh:HEADS + hh + 1]
                  * edT_ref[HEADS + hh:HEADS + hh + 1, :]).astype(jnp.bfloat16)
            u = jnp.maximum(u1, u2) * adj
            oz = jax.lax.dot_general(
                u, Whz_ref[hh],
                (((1,), (0,)), ((), ())),
                preferred_element_type=jnp.float32)
            outs.append(oz[:, 0:dh] / oz[:, dh:dh + 1])
        if concat:
            out = jnp.concatenate(outs, axis=1)
        else:
            out = sum(outs) * (1.0 / HEADS)
        out = jnp.where(out > 0, out, jnp.exp(out) - 1.0)
        if residual:
            out = out + h_ref[pl.ds(i * BN, BN), :]
        out_ref[...] = out

    return pl.pallas_call(
        body,
        grid=(n // BN,),
        in_specs=[
            pl.BlockSpec((n, din), lambda i: (0, 0)),
            pl.BlockSpec((BN, n), lambda i: (i, 0)),
            pl.BlockSpec((din, hd), lambda i: (0, 0)),
            pl.BlockSpec((hd, HEADS), lambda i: (0, 0)),
            pl.BlockSpec((hd, HEADS), lambda i: (0, 0)),
        ],
        out_specs=pl.BlockSpec((BN, dout), lambda i: (i, 0)),
        out_shape=jax.ShapeDtypeStruct((n, dout), jnp.float32),
        scratch_shapes=[
            pltpu.VMEM((n, hd), jnp.float32),
            pltpu.VMEM((HEADS, n, dh + 1), jnp.bfloat16),
            pltpu.VMEM((n, 2 * HEADS), jnp.float32),
            pltpu.VMEM((2 * HEADS, n), jnp.float32),
        ],
    )(h, adj, W, Asrc, Adst)


def _head(h, cls2, W_out, b_out2):
    n, dm = h.shape
    B = cls2.shape[0]
    nclass = W_out.shape[1]

    def body(h_ref, cls_ref, Wo_ref, bo_ref, out_ref):
        ids = cls_ref[...]
        iota = jax.lax.broadcasted_iota(jnp.int32, (B, n), 1)
        onehot = (iota == ids).astype(jnp.float32)
        cls_h = jnp.dot(onehot, h_ref[...], preferred_element_type=jnp.float32)
        out_ref[...] = jnp.dot(cls_h, Wo_ref[...],
                               preferred_element_type=jnp.float32) + bo_ref[...]

    return pl.pallas_call(
        body,
        out_shape=jax.ShapeDtypeStruct((B, nclass), jnp.float32),
    )(h, cls2, W_out, b_out2)


_LOG2E = 1.4426950408889634


def _blockdiag(a):
    H, dh = a.shape
    eye = jnp.eye(H, dtype=a.dtype)
    return (a[:, :, None] * eye[:, None, :]).reshape(H * dh, H) * _LOG2E


def kernel(word_ids, adj, edge_type, cls_node, emb, W_mid, a_src_mid,
           a_dst_mid, W_last, a_src_last, a_dst_last, W_out, b_out):
    h = _embed_gather(emb, word_ids.astype(jnp.int32))
    adj = adj.astype(jnp.bfloat16)
    for l in range(W_mid.shape[0]):
        h = _gat_layer(h, adj, W_mid[l],
                       _blockdiag(a_src_mid[l]), _blockdiag(a_dst_mid[l]),
                       dh=32, concat=True, residual=True)
    h = _gat_layer(h, adj, W_last,
                   _blockdiag(a_src_last), _blockdiag(a_dst_last),
                   dh=D, concat=False, residual=False)
    logits = _head(h, cls_node.astype(jnp.int32).reshape(-1, 1),
                   W_out, b_out.reshape(1, -1))
    return (logits,)


# speedup vs baseline: 1.0148x; 1.0148x over previous
import functools

import jax
import jax.numpy as jnp
from jax.experimental import pallas as pl
from jax.experimental.pallas import tpu as pltpu
from jax.experimental.pallas import tpu_sc as plsc

N = 4096
D = 128
HEADS = 4


def _embed_gather(emb, ids):
    V, Dm = emb.shape
    B = ids.shape[0]
    info = plsc.get_sparse_core_info()
    NC, NS = info.num_cores, info.num_subcores
    NW = NC * NS
    bpw = B // NW
    mesh = plsc.VectorSubcoreMesh(core_axis_name="c", subcore_axis_name="s")

    @functools.partial(
        pl.kernel,
        mesh=mesh,
        out_type=jax.ShapeDtypeStruct((B, Dm), jnp.float32),
        scratch_types=[
            pltpu.VMEM((bpw,), jnp.int32),
            pltpu.VMEM((bpw, Dm), jnp.float32),
            pltpu.SemaphoreType.DMA,
        ],
    )
    def gather_kernel(emb_hbm, ids_hbm, out_hbm, idx_v, rows_v, sem):
        wid = jax.lax.axis_index("s") * NC + jax.lax.axis_index("c")
        base = wid * bpw
        pltpu.sync_copy(ids_hbm.at[pl.ds(base, bpw)], idx_v)
        pltpu.async_copy(emb_hbm.at[idx_v], rows_v, sem).wait()
        pltpu.sync_copy(rows_v, out_hbm.at[pl.ds(base, bpw)])

    return gather_kernel(emb, ids)


def _gat_layer(h, adj, W, Asrc, Adst, dh, concat, residual):
    n, din = h.shape
    hd = HEADS * dh
    dout = hd if concat else dh
    BN = 512 if concat else 256

    def body(h_ref, adj_ref, W_ref, Asrc_ref, Adst_ref, out_ref,
             Wh_ref, Whz_ref, es_ref, edT_ref):
        i = pl.program_id(0)

        @pl.when(i == 0)
        def _():
            Wh = jnp.dot(h_ref[...], W_ref[...],
                         preferred_element_type=jnp.float32)
            Wh_ref[...] = Wh
            for g in range(HEADS):
                Whz_ref[g] = jnp.concatenate(
                    [Wh[:, g * dh:(g + 1) * dh].astype(jnp.bfloat16),
                     jnp.ones((n, 1), jnp.bfloat16)], axis=1)
            src = jnp.dot(Wh, Asrc_ref[...],
                          preferred_element_type=jnp.float32)
            dstT = jnp.dot(Wh, Adst_ref[...],
                           preferred_element_type=jnp.float32).T
            es_ref[...] = jnp.exp2(
                jnp.concatenate([src, 0.2 * src], axis=1))
            edT_ref[...] = jnp.exp2(
                jnp.concatenate([dstT, 0.2 * dstT], axis=0))

        adj = adj_ref[...]
        esB = es_ref[pl.ds(i * BN, BN), :]
        outs = []
        for hh in range(HEADS):
            u1 = (esB[:, hh:hh + 1]
                  * edT_ref[hh:hh + 1, :]).astype(jnp.bfloat16)
            u2 = (esB[:, HEADS + hh:HEADS + hh + 1]
                  * edT_ref[HEADS + hh:HEADS + hh + 1, :]).astype(jnp.bfloat16)
            u = jnp.maximum(u1, u2) * adj
            oz = jax.lax.dot_general(
                u, Whz_ref[hh],
                (((1,), (0,)), ((), ())),
                preferred_element_type=jnp.float32)
            outs.append(oz[:, 0:dh] / oz[:, dh:dh + 1])
        if concat:
            out = jnp.concatenate(outs, axis=1)
        else:
            out = sum(outs) * (1.0 / HEADS)
        out = jnp.where(out > 0, out, jnp.exp(out) - 1.0)
        if residual:
            out = out + h_ref[pl.ds(i * BN, BN), :]
        out_ref[...] = out

    return pl.pallas_call(
        body,
        grid=(n // BN,),
        in_specs=[
            pl.BlockSpec((n, din), lambda i: (0, 0)),
            pl.BlockSpec((BN, n), lambda i: (i, 0)),
            pl.BlockSpec((din, hd), lambda i: (0, 0)),
            pl.BlockSpec((hd, HEADS), lambda i: (0, 0)),
            pl.BlockSpec((hd, HEADS), lambda i: (0, 0)),
        ],
        out_specs=pl.BlockSpec((BN, dout), lambda i: (i, 0)),
        out_shape=jax.ShapeDtypeStruct((n, dout), jnp.float32),
        scratch_shapes=[
            pltpu.VMEM((n, hd), jnp.float32),
            pltpu.VMEM((HEADS, n, dh + 1), jnp.bfloat16),
            pltpu.VMEM((n, 2 * HEADS), jnp.float32),
            pltpu.VMEM((2 * HEADS, n), jnp.float32),
        ],
    )(h, adj, W, Asrc, Adst)


def _head(h, cls2, W_out, b_out2):
    n, dm = h.shape
    B = cls2.shape[0]
    nclass = W_out.shape[1]

    def body(h_ref, cls_ref, Wo_ref, bo_ref, out_ref):
        ids = cls_ref[...]
        iota = jax.lax.broadcasted_iota(jnp.int32, (B, n), 1)
        onehot = (iota == ids).astype(jnp.float32)
        cls_h = jnp.dot(onehot, h_ref[...], preferred_element_type=jnp.float32)
        out_ref[...] = jnp.dot(cls_h, Wo_ref[...],
                               preferred_element_type=jnp.float32) + bo_ref[...]

    return pl.pallas_call(
        body,
        out_shape=jax.ShapeDtypeStruct((B, nclass), jnp.float32),
    )(h, cls2, W_out, b_out2)


_LOG2E = 1.4426950408889634


def _blockdiag(a):
    H, dh = a.shape
    eye = jnp.eye(H, dtype=a.dtype)
    return (a[:, :, None] * eye[:, None, :]).reshape(H * dh, H) * _LOG2E


def kernel(word_ids, adj, edge_type, cls_node, emb, W_mid, a_src_mid,
           a_dst_mid, W_last, a_src_last, a_dst_last, W_out, b_out):
    h = _embed_gather(emb, word_ids.astype(jnp.int32))
    adj = adj.astype(jnp.bfloat16)
    for l in range(W_mid.shape[0]):
        h = _gat_layer(h, adj, W_mid[l],
                       _blockdiag(a_src_mid[l]), _blockdiag(a_dst_mid[l]),
                       dh=32, concat=True, residual=True)
    h = _gat_layer(h, adj, W_last,
                   _blockdiag(a_src_last), _blockdiag(a_dst_last),
                   dh=D, concat=False, residual=False)
    logits = _head(h, cls_node.astype(jnp.int32).reshape(-1, 1),
                   W_out, b_out.reshape(1, -1))
    return (logits,)


# speedup vs baseline: 1.0695x; 1.0539x over previous
import functools

import jax
import jax.numpy as jnp
from jax.experimental import pallas as pl
from jax.experimental.pallas import tpu as pltpu
from jax.experimental.pallas import tpu_sc as plsc

N = 4096
D = 128
HEADS = 4
DHM = 32
BN = 512
_LOG2E = 1.4426950408889634


def _embed_gather(emb, ids):
    V, Dm = emb.shape
    B = ids.shape[0]
    info = plsc.get_sparse_core_info()
    NC, NS = info.num_cores, info.num_subcores
    NW = NC * NS
    bpw = B // NW
    mesh = plsc.VectorSubcoreMesh(core_axis_name="c", subcore_axis_name="s")

    @functools.partial(
        pl.kernel,
        mesh=mesh,
        out_type=jax.ShapeDtypeStruct((B, Dm), jnp.float32),
        scratch_types=[
            pltpu.VMEM((bpw,), jnp.int32),
            pltpu.VMEM((bpw, Dm), jnp.float32),
            pltpu.SemaphoreType.DMA,
        ],
    )
    def gather_kernel(emb_hbm, ids_hbm, out_hbm, idx_v, rows_v, sem):
        wid = jax.lax.axis_index("s") * NC + jax.lax.axis_index("c")
        base = wid * bpw
        pltpu.sync_copy(ids_hbm.at[pl.ds(base, bpw)], idx_v)
        pltpu.async_copy(emb_hbm.at[idx_v], rows_v, sem).wait()
        pltpu.sync_copy(rows_v, out_hbm.at[pl.ds(base, bpw)])

    return gather_kernel(emb, ids)


def _gat_forward(h0, adj, Wstk, Astk, cls2, W_out, b_out2):
    n = N
    nblk = n // BN

    def body(h0_ref, adj_ref, Wstk_ref, Astk_ref, cls_ref, Wo_ref, bo_ref,
             out_ref, hA_ref, hB_ref, WhzM_ref, WhzL_ref,
             es_ref, edT_ref):
        l = pl.program_id(0)
        i = pl.program_id(1)

        def prep(hin, dh, hd):
            Wh = jnp.dot(hin, Wstk_ref[0][:, 0:hd],
                         preferred_element_type=jnp.float32)
            Whz_ref = WhzM_ref if dh == DHM else WhzL_ref
            for g in range(HEADS):
                Whz_ref[g] = jnp.concatenate(
                    [Wh[:, g * dh:(g + 1) * dh].astype(jnp.bfloat16),
                     jnp.ones((n, 1), jnp.bfloat16)], axis=1)
            src = jnp.dot(Wh, Astk_ref[0][0:hd, 0:HEADS],
                          preferred_element_type=jnp.float32)
            dstT = jnp.dot(Wh, Astk_ref[0][0:hd, HEADS:2 * HEADS],
                           preferred_element_type=jnp.float32).T
            es_ref[...] = jnp.exp2(
                jnp.concatenate([src, 0.2 * src], axis=1))
            edT_ref[...] = jnp.exp2(
                jnp.concatenate([dstT, 0.2 * dstT], axis=0))

        def attend(dh, concat):
            adjb = adj_ref[...]
            esB = es_ref[pl.ds(i * BN, BN), :]
            Whz_ref = WhzM_ref if dh == DHM else WhzL_ref
            outs = []
            for hh in range(HEADS):
                u1 = (esB[:, hh:hh + 1]
                      * edT_ref[hh:hh + 1, :]).astype(jnp.bfloat16)
                u2 = (esB[:, HEADS + hh:HEADS + hh + 1]
                      * edT_ref[HEADS + hh:HEADS + hh + 1, :]
                      ).astype(jnp.bfloat16)
                u = jnp.maximum(u1, u2) * adjb
                oz = jax.lax.dot_general(
                    u, Whz_ref[hh],
                    (((1,), (0,)), ((), ())),
                    preferred_element_type=jnp.float32)
                outs.append(oz[:, 0:dh] / oz[:, dh:dh + 1])
            if concat:
                out = jnp.concatenate(outs, axis=1)
            else:
                out = sum(outs) * (1.0 / HEADS)
            return jnp.where(out > 0, out, jnp.exp(out) - 1.0)

        @pl.when(l == 0)
        def _():
            @pl.when(i == 0)
            def _():
                prep(h0_ref[...], DHM, D)
            hA_ref[pl.ds(i * BN, BN), :] = \
                attend(DHM, True) + h0_ref[pl.ds(i * BN, BN), :]

        @pl.when(l == 1)
        def _():
            @pl.when(i == 0)
            def _():
                prep(hA_ref[...], DHM, D)
            hB_ref[pl.ds(i * BN, BN), :] = \
                attend(DHM, True) + hA_ref[pl.ds(i * BN, BN), :]

        @pl.when(l == 2)
        def _():
            @pl.when(i == 0)
            def _():
                prep(hB_ref[...], D, HEADS * D)
            hA_ref[pl.ds(i * BN, BN), :] = attend(D, False)

            @pl.when(i == nblk - 1)
            def _():
                ids = cls_ref[...]
                iota = jax.lax.broadcasted_iota(
                    jnp.int32, (ids.shape[0], n), 1)
                onehot = (iota == ids).astype(jnp.float32)
                cls_h = jnp.dot(onehot, hA_ref[...],
                                preferred_element_type=jnp.float32)
                out_ref[...] = jnp.dot(
                    cls_h, Wo_ref[...],
                    preferred_element_type=jnp.float32) + bo_ref[...]

    B = cls2.shape[0]
    nclass = W_out.shape[1]
    return pl.pallas_call(
        body,
        grid=(3, nblk),
        in_specs=[
            pl.BlockSpec((n, D), lambda l, i: (0, 0)),
            pl.BlockSpec((BN, n), lambda l, i: (i, 0)),
            pl.BlockSpec((1, D, HEADS * D), lambda l, i: (l, 0, 0)),
            pl.BlockSpec((1, HEADS * D, 2 * HEADS),
                         lambda l, i: (l, 0, 0)),
            pl.BlockSpec((B, 1), lambda l, i: (0, 0)),
            pl.BlockSpec((D, nclass), lambda l, i: (0, 0)),
            pl.BlockSpec((1, nclass), lambda l, i: (0, 0)),
        ],
        out_specs=pl.BlockSpec((B, nclass), lambda l, i: (0, 0)),
        out_shape=jax.ShapeDtypeStruct((B, nclass), jnp.float32),
        scratch_shapes=[
            pltpu.VMEM((n, D), jnp.float32),
            pltpu.VMEM((n, D), jnp.float32),
            pltpu.VMEM((HEADS, n, DHM + 1), jnp.bfloat16),
            pltpu.VMEM((HEADS, n, D + 1), jnp.bfloat16),
            pltpu.VMEM((n, 2 * HEADS), jnp.float32),
            pltpu.VMEM((2 * HEADS, n), jnp.float32),
        ],
    )(h0, adj, Wstk, Astk, cls2, W_out, b_out2)


def _blockdiag(a, hd):
    H, dh = a.shape
    eye = jnp.eye(H, dtype=a.dtype)
    bd = (a[:, :, None] * eye[:, None, :]).reshape(H * dh, H) * _LOG2E
    return jnp.pad(bd, ((0, hd - H * dh), (0, 0)))


def kernel(word_ids, adj, edge_type, cls_node, emb, W_mid, a_src_mid,
           a_dst_mid, W_last, a_src_last, a_dst_last, W_out, b_out):
    h0 = _embed_gather(emb, word_ids.astype(jnp.int32))
    hd = HEADS * D
    Wstk = jnp.stack([
        jnp.pad(W_mid[0], ((0, 0), (0, hd - D))),
        jnp.pad(W_mid[1], ((0, 0), (0, hd - D))),
        W_last,
    ])
    Astk = jnp.stack([
        jnp.concatenate([_blockdiag(a_src_mid[0], hd),
                         _blockdiag(a_dst_mid[0], hd)], axis=1),
        jnp.concatenate([_blockdiag(a_src_mid[1], hd),
                         _blockdiag(a_dst_mid[1], hd)], axis=1),
        jnp.concatenate([_blockdiag(a_src_last, hd),
                         _blockdiag(a_dst_last, hd)], axis=1),
    ])
    logits = _gat_forward(h0, adj.astype(jnp.bfloat16), Wstk, Astk,
                          cls_node.astype(jnp.int32).reshape(-1, 1),
                          W_out, b_out.reshape(1, -1))
    return (logits,)
